# baseline (speedup 1.0000x reference)
import functools

import jax
import jax.numpy as jnp
import numpy as np
from jax.experimental import pallas as pl
from jax.experimental.pallas import tpu as pltpu

_BEV_H, _BEV_W, _BEV_Z = 128, 128, 16
_EMBED = 128
_N_VOX = _BEV_H * _BEV_W * _BEV_Z
_SCENE = np.array([51.2, 51.2, 6.4])
_PC_RANGE = np.array([0.0, -25.6, -2.0, 51.2, 25.6, 4.4])

_BN = 8192
_JT = _BN // 128
_BH = _BN // (_BEV_W * _BEV_Z)


def _ref3d_rows() -> np.ndarray:
    voxel_size = _SCENE[0] / _BEV_H
    vol_dim = np.ceil(_SCENE / voxel_size).astype(int)
    xv, yv, zv = np.meshgrid(np.arange(vol_dim[0]), np.arange(vol_dim[1]),
                             np.arange(vol_dim[2]), indexing='ij')
    r3d = np.stack([(yv.reshape(-1) + 0.5) / _BEV_W,
                    (xv.reshape(-1) + 0.5) / _BEV_H,
                    (zv.reshape(-1) + 0.5) / _BEV_Z], axis=1).astype(np.float64)
    r3d[:, 0] = r3d[:, 0] * (_PC_RANGE[3] - _PC_RANGE[0]) + _PC_RANGE[0]
    r3d[:, 1] = r3d[:, 1] * (_PC_RANGE[4] - _PC_RANGE[1]) + _PC_RANGE[1]
    r3d[:, 2] = r3d[:, 2] * (_PC_RANGE[5] - _PC_RANGE[2]) + _PC_RANGE[2]
    return np.ascontiguousarray(r3d.T.astype(np.float32)).reshape(
        3, _N_VOX // 128, 128)


_R3D_T = _ref3d_rows()


def _ctx_kernel(img_ref, ctx_ref):
    ones = jnp.full((1, img_ref.shape[0]), 1.0 / img_ref.shape[0],
                    dtype=jnp.float32)
    ctx_ref[...] = jax.lax.dot_general(
        ones, img_ref[...], (((1,), (0,)), ((), ())),
        preferred_element_type=jnp.float32)


def _main_kernel(embed_ref, vf_ref, anchor_ref, r3d_ref, wcat_ref, ctx_ref,
                 out_ref):
    w = wcat_ref[:, :_EMBED]
    ut = jax.lax.dot_general(w, embed_ref[...], (((0,), (1,)), ((), ())),
                             preferred_element_type=jnp.float32)
    ctx_col = ctx_ref[...].T
    lane = jax.lax.broadcasted_iota(jnp.int32, (2 * _EMBED, 128), 1)
    src = (lane % _BEV_Z) * 8 + lane // _BEV_Z
    out_ref[0:3, :, :] = r3d_ref[...]
    for s in range(_BH):
        x_s = vf_ref[:, s]
        vw = jax.lax.dot_general(x_s, wcat_ref[...], (((0,), (0,)), ((), ())),
                                 preferred_element_type=jnp.float32)
        for j in range(_BEV_Z):
            jj = s * _BEV_Z + j
            blk = vw[:, 8 * j:8 * j + 8, :].reshape(128, 2 * _EMBED)
            bt = jnp.take_along_axis(blk.T, src, axis=1)
            tj = jnp.tanh(ut[:, jj * 128:(jj + 1) * 128] + bt[:_EMBED])
            tj = tj + ctx_col
            mrow = anchor_ref[jj:jj + 1, :] > 0
            out_ref[3:, jj, :] = jnp.where(mrow, tj, bt[_EMBED:])


@functools.partial(jax.jit, static_argnames=("interpret",))
def _run(mlvl_feats, voxel_feat, voxel_anchor, voxel_embed, W_refine,
         interpret=False):
    img2d = mlvl_feats[0, 0].transpose(1, 2, 0).reshape(-1, _EMBED)
    ctx = pl.pallas_call(
        _ctx_kernel,
        out_shape=jax.ShapeDtypeStruct((1, _EMBED), jnp.float32),
        interpret=interpret,
    )(img2d)

    vf4 = voxel_feat[0].transpose(0, 1, 3, 2)
    anchor2 = voxel_anchor.reshape(_N_VOX // 128, 128)
    r3d = jnp.asarray(_R3D_T)
    wcat = jnp.concatenate(
        [W_refine, jnp.eye(_EMBED, dtype=jnp.float32)], axis=1)
    grid = (_N_VOX // _BN,)
    out = pl.pallas_call(
        _main_kernel,
        grid=grid,
        in_specs=[
            pl.BlockSpec((_BN, _EMBED), lambda i: (i, 0)),
            pl.BlockSpec((_EMBED, _BH, _BEV_Z, _BEV_W),
                         lambda i: (0, i, 0, 0)),
            pl.BlockSpec((_JT, 128), lambda i: (i, 0)),
            pl.BlockSpec((3, _JT, 128), lambda i: (0, i, 0)),
            pl.BlockSpec((_EMBED, 2 * _EMBED), lambda i: (0, 0)),
            pl.BlockSpec((1, _EMBED), lambda i: (0, 0)),
        ],
        out_specs=pl.BlockSpec((_EMBED + 3, _JT, 128), lambda i: (0, i, 0)),
        out_shape=jax.ShapeDtypeStruct((_EMBED + 3, _N_VOX // 128, 128),
                                       jnp.float32),
        compiler_params=pltpu.CompilerParams(
            dimension_semantics=("parallel",)),
        interpret=interpret,
    )(voxel_embed, vf4, anchor2, r3d, wcat, ctx)
    return out.reshape(1, _EMBED + 3, _N_VOX)


def kernel(mlvl_feats, voxel_feat, voxel_anchor, voxel_embed, W_refine,
           cam_params, img_metas):
    return _run(mlvl_feats, voxel_feat, voxel_anchor, voxel_embed, W_refine)


# speedup vs baseline: 1.0415x; 1.0415x over previous
import functools

import jax
import jax.numpy as jnp
import numpy as np
from jax.experimental import pallas as pl
from jax.experimental.pallas import tpu as pltpu

_BEV_H, _BEV_W, _BEV_Z = 128, 128, 16
_EMBED = 128
_N_VOX = _BEV_H * _BEV_W * _BEV_Z
_SCENE = np.array([51.2, 51.2, 6.4])
_PC_RANGE = np.array([0.0, -25.6, -2.0, 51.2, 25.6, 4.4])

_BN = 8192
_JT = _BN // 128


def _ref3d_rows() -> np.ndarray:
    voxel_size = _SCENE[0] / _BEV_H
    vol_dim = np.ceil(_SCENE / voxel_size).astype(int)
    xv, yv, zv = np.meshgrid(np.arange(vol_dim[0]), np.arange(vol_dim[1]),
                             np.arange(vol_dim[2]), indexing='ij')
    r3d = np.stack([(yv.reshape(-1) + 0.5) / _BEV_W,
                    (xv.reshape(-1) + 0.5) / _BEV_H,
                    (zv.reshape(-1) + 0.5) / _BEV_Z], axis=1).astype(np.float64)
    r3d[:, 0] = r3d[:, 0] * (_PC_RANGE[3] - _PC_RANGE[0]) + _PC_RANGE[0]
    r3d[:, 1] = r3d[:, 1] * (_PC_RANGE[4] - _PC_RANGE[1]) + _PC_RANGE[1]
    r3d[:, 2] = r3d[:, 2] * (_PC_RANGE[5] - _PC_RANGE[2]) + _PC_RANGE[2]
    return np.ascontiguousarray(r3d.T.astype(np.float32)).reshape(
        3, _N_VOX // 128, 128)


_R3D_T = _ref3d_rows()


def _ctx_kernel(img_ref, ctx_ref):
    ones = jnp.full((1, img_ref.shape[0]), 1.0 / img_ref.shape[0],
                    dtype=jnp.float32)
    ctx_ref[...] = jax.lax.dot_general(
        ones, img_ref[...], (((1,), (0,)), ((), ())),
        preferred_element_type=jnp.float32)


def _main_kernel(embed_ref, vf_ref, anchor_ref, r3d_ref, w_ref, ctx_ref,
                 out_ref):
    e = embed_ref[...]
    v = vf_ref[...]
    t = jax.lax.dot_general(e + v, w_ref[...], (((1,), (0,)), ((), ())),
                            preferred_element_type=jnp.float32)
    t = jnp.tanh(t) + ctx_ref[...]
    m = anchor_ref[...] > 0
    sel = jnp.where(m, t, v)
    out_ref[0:3, :, :] = r3d_ref[...]
    for j in range(_JT):
        out_ref[3:, j, :] = sel[j * 128:(j + 1) * 128, :].T


@functools.partial(jax.jit, static_argnames=("interpret",))
def _run(mlvl_feats, voxel_feat, voxel_anchor, voxel_embed, W_refine,
         interpret=False):
    img2d = mlvl_feats[0, 0].transpose(1, 2, 0).reshape(-1, _EMBED)
    ctx = pl.pallas_call(
        _ctx_kernel,
        out_shape=jax.ShapeDtypeStruct((1, _EMBED), jnp.float32),
        interpret=interpret,
    )(img2d)

    vf_nc = voxel_feat[0].transpose(1, 2, 3, 0).reshape(_N_VOX, _EMBED)
    anchor_col = voxel_anchor.reshape(_N_VOX, 1)
    r3d = jnp.asarray(_R3D_T)
    grid = (_N_VOX // _BN,)
    out = pl.pallas_call(
        _main_kernel,
        grid=grid,
        in_specs=[
            pl.BlockSpec((_BN, _EMBED), lambda i: (i, 0)),
            pl.BlockSpec((_BN, _EMBED), lambda i: (i, 0)),
            pl.BlockSpec((_BN, 1), lambda i: (i, 0)),
            pl.BlockSpec((3, _JT, 128), lambda i: (0, i, 0)),
            pl.BlockSpec((_EMBED, _EMBED), lambda i: (0, 0)),
            pl.BlockSpec((1, _EMBED), lambda i: (0, 0)),
        ],
        out_specs=pl.BlockSpec((_EMBED + 3, _JT, 128), lambda i: (0, i, 0)),
        out_shape=jax.ShapeDtypeStruct((_EMBED + 3, _N_VOX // 128, 128),
                                       jnp.float32),
        compiler_params=pltpu.CompilerParams(
            dimension_semantics=("parallel",)),
        interpret=interpret,
    )(voxel_embed, vf_nc, anchor_col, r3d, W_refine, ctx)
    return out.reshape(1, _EMBED + 3, _N_VOX)


def kernel(mlvl_feats, voxel_feat, voxel_anchor, voxel_embed, W_refine,
           cam_params, img_metas):
    return _run(mlvl_feats, voxel_feat, voxel_anchor, voxel_embed, W_refine)


# speedup vs baseline: 1.0427x; 1.0011x over previous
import functools

import jax
import jax.numpy as jnp
import numpy as np
from jax.experimental import pallas as pl
from jax.experimental.pallas import tpu as pltpu

_BEV_H, _BEV_W, _BEV_Z = 128, 128, 16
_EMBED = 128
_N_VOX = _BEV_H * _BEV_W * _BEV_Z
_SCENE = np.array([51.2, 51.2, 6.4])
_PC_RANGE = np.array([0.0, -25.6, -2.0, 51.2, 25.6, 4.4])

_BN = 8192
_JT = _BN // 128


def _ref3d_rows() -> np.ndarray:
    voxel_size = _SCENE[0] / _BEV_H
    vol_dim = np.ceil(_SCENE / voxel_size).astype(int)
    xv, yv, zv = np.meshgrid(np.arange(vol_dim[0]), np.arange(vol_dim[1]),
                             np.arange(vol_dim[2]), indexing='ij')
    r3d = np.stack([(yv.reshape(-1) + 0.5) / _BEV_W,
                    (xv.reshape(-1) + 0.5) / _BEV_H,
                    (zv.reshape(-1) + 0.5) / _BEV_Z], axis=1).astype(np.float64)
    r3d[:, 0] = r3d[:, 0] * (_PC_RANGE[3] - _PC_RANGE[0]) + _PC_RANGE[0]
    r3d[:, 1] = r3d[:, 1] * (_PC_RANGE[4] - _PC_RANGE[1]) + _PC_RANGE[1]
    r3d[:, 2] = r3d[:, 2] * (_PC_RANGE[5] - _PC_RANGE[2]) + _PC_RANGE[2]
    return np.ascontiguousarray(r3d.T.astype(np.float32)).reshape(
        3, _N_VOX // 128, 128)


_R3D_T = _ref3d_rows()


def _ctx_kernel(img_ref, ctx_ref):
    ones = jnp.full((1, img_ref.shape[0]), 1.0 / img_ref.shape[0],
                    dtype=jnp.float32)
    ctx_ref[...] = jax.lax.dot_general(
        ones, img_ref[...], (((1,), (0,)), ((), ())),
        preferred_element_type=jnp.float32)


def _main_kernel(embed_ref, vf_ref, anchor_ref, r3d_ref, w_ref, ctx_ref,
                 out_ref):
    e = embed_ref[...]
    v = vf_ref[...]
    t = jax.lax.dot_general(e + v, w_ref[...], (((1,), (0,)), ((), ())),
                            preferred_element_type=jnp.float32)
    t = jnp.tanh(t) + ctx_ref[...]
    m = anchor_ref[...] > 0
    sel = jnp.where(m, t, v)
    out_ref[0:3, :, :] = r3d_ref[...]
    for j in range(_JT):
        out_ref[3:, j, :] = sel[j * 128:(j + 1) * 128, :].T


@functools.partial(jax.jit, static_argnames=("interpret",))
def _run(mlvl_feats, voxel_feat, voxel_anchor, voxel_embed, W_refine,
         interpret=False):
    img2d = mlvl_feats[0, 0].transpose(1, 2, 0).reshape(-1, _EMBED)
    ctx = pl.pallas_call(
        _ctx_kernel,
        out_shape=jax.ShapeDtypeStruct((1, _EMBED), jnp.float32),
        interpret=interpret,
    )(img2d)

    vf_nc = voxel_feat[0].transpose(1, 2, 3, 0).reshape(_N_VOX, _EMBED)
    anchor_col = voxel_anchor.reshape(_N_VOX, 1)
    r3d = jnp.asarray(_R3D_T)
    grid = (_N_VOX // _BN,)
    out = pl.pallas_call(
        _main_kernel,
        grid=grid,
        in_specs=[
            pl.BlockSpec((_BN, _EMBED), lambda i: (i, 0)),
            pl.BlockSpec((_BN, _EMBED), lambda i: (i, 0)),
            pl.BlockSpec((_BN, 1), lambda i: (i, 0)),
            pl.BlockSpec((3, _JT, 128), lambda i: (0, i, 0)),
            pl.BlockSpec((_EMBED, _EMBED), lambda i: (0, 0)),
            pl.BlockSpec((1, _EMBED), lambda i: (0, 0)),
        ],
        out_specs=pl.BlockSpec((_EMBED + 3, _JT, 128), lambda i: (0, i, 0)),
        out_shape=jax.ShapeDtypeStruct((_EMBED + 3, _N_VOX // 128, 128),
                                       jnp.float32),
        compiler_params=pltpu.CompilerParams(
            dimension_semantics=("arbitrary",)),
        interpret=interpret,
    )(voxel_embed, vf_nc, anchor_col, r3d, W_refine, ctx)
    return out.reshape(1, _EMBED + 3, _N_VOX)


def kernel(mlvl_feats, voxel_feat, voxel_anchor, voxel_embed, W_refine,
           cam_params, img_metas):
    return _run(mlvl_feats, voxel_feat, voxel_anchor, voxel_embed, W_refine)
